# speedup vs baseline: 6.2684x; 6.2684x over previous
import functools

import jax
import jax.numpy as jnp
from jax.experimental import pallas as pl

_EPS = 1e-12


def _body(ids_ref, sid_ref, char_ref, spk_ref, pos_ref, g_ref, b_ref, o_ref,
          *, bb, seq, vocab, hid, nspk):
    ids = ids_ref[...]
    iota_v = jax.lax.broadcasted_iota(jnp.int32, (bb, seq, vocab), 2)
    oh = (ids[:, :, None] == iota_v).astype(jnp.float32)
    oh = oh.reshape(bb * seq, vocab).astype(jnp.bfloat16)
    emb = jax.lax.dot_general(
        oh, char_ref[...], (((1,), (0,)), ((), ())),
        preferred_element_type=jnp.float32)
    x = emb.reshape(bb, seq, hid) + pos_ref[...][None]

    sid = sid_ref[0, 0, :]
    iota_n = jax.lax.broadcasted_iota(jnp.int32, (bb, nspk), 1)
    m = (sid[:, None] == iota_n).astype(jnp.float32)
    spk_sel = jnp.sum(m[:, :, None] * spk_ref[...][None, :, :], axis=1)
    x = x + spk_sel[:, None, :]

    mu = jnp.mean(x, axis=-1, keepdims=True)
    xc = x - mu
    var = jnp.mean(xc * xc, axis=-1, keepdims=True)
    y = xc * jax.lax.rsqrt(var + _EPS)
    o_ref[...] = y * g_ref[...][None] + b_ref[...][None]


def kernel(input_ids, speaker_ids, char_emb, spk_emb, pos_emb, gamma, beta):
    batch, seq = input_ids.shape
    vocab, hid = char_emb.shape
    nspk = spk_emb.shape[0]
    bb = 16
    nblk = batch // bb

    char_bf = char_emb.astype(jnp.bfloat16)
    pos_s = pos_emb[:seq]
    sids = speaker_ids.reshape(nblk, 1, bb)

    body = functools.partial(_body, bb=bb, seq=seq, vocab=vocab, hid=hid,
                             nspk=nspk)

    return pl.pallas_call(
        body,
        grid=(nblk,),
        in_specs=[
            pl.BlockSpec((bb, seq), lambda i: (i, 0)),
            pl.BlockSpec((1, 1, bb), lambda i: (i, 0, 0)),
            pl.BlockSpec((vocab, hid), lambda i: (0, 0)),
            pl.BlockSpec((nspk, hid), lambda i: (0, 0)),
            pl.BlockSpec((seq, hid), lambda i: (0, 0)),
            pl.BlockSpec((1, hid), lambda i: (0, 0)),
            pl.BlockSpec((1, hid), lambda i: (0, 0)),
        ],
        out_specs=pl.BlockSpec((bb, seq, hid), lambda i: (i, 0, 0)),
        out_shape=jax.ShapeDtypeStruct((batch, seq, hid), jnp.float32),
    )(input_ids, sids, char_bf, spk_emb, pos_s,
      gamma.reshape(1, hid), beta.reshape(1, hid))
